# baseline (speedup 1.0000x reference)
import jax
import jax.numpy as jnp
from jax.experimental import pallas as pl
from jax.experimental.pallas import tpu as pltpu
from jax.experimental.pallas import tpu_sc as plsc

P_ROWS = 4096
N_ROWS = 200
D_MODEL = 256
D8 = D_MODEL // 8
D_REL = 64
NUM_IDX = P_ROWS * N_ROWS
GATHER_W = 128

_OFF_M, _OFF_W, _OFF_H, _OFF_MIN, _OFF_S = 0, 12, 17, 41, 101
_COMB = 105
_COMB_PAD = 128


def _relidx_body(pts_ref, nts_t_ref, out_ref):
    mb = pts_ref[:, 2:3] * 60 + pts_ref[:, 3:4]
    ma = nts_t_ref[2:3, :] * 60 + nts_t_ref[3:4, :]
    out_ref[...] = jnp.clip(mb - ma, -500, 500) + 500


def _session_col(hour, minute):
    t = hour * 60 + minute
    return jnp.where(t < 4 * 60, 0,
           jnp.where(t < 9 * 60 + 30, 1,
           jnp.where(t < 16 * 60, 2,
           jnp.where(t < 20 * 60, 3, 0))))


def _embed_body(pts_ref, nts_ref, bdiag_ref, w_ref, b_ref, scale_ref,
                pout_ref, nout_ref):
    bdiag = bdiag_ref[...]
    w = w_ref[...]
    bias = b_ref[...]

    def emb(ts, nrows, scale_val):
        cm = ts[:, 0:1] - 1 + _OFF_M
        cw = ts[:, 1:2] + _OFF_W
        ch = ts[:, 2:3] + _OFF_H
        cmin = ts[:, 3:4] + _OFF_MIN
        cs = _session_col(ts[:, 2:3], ts[:, 3:4]) + _OFF_S
        col = jax.lax.broadcasted_iota(jnp.int32, (nrows, _COMB_PAD), 1)
        h = ((col == cm).astype(jnp.float32)
             + (col == cw).astype(jnp.float32)
             + (col == ch).astype(jnp.float32)
             + (col == cmin).astype(jnp.float32)
             + (col == cs).astype(jnp.float32))
        feats = jnp.dot(h, bdiag, preferred_element_type=jnp.float32)
        out = jnp.dot(feats, w, preferred_element_type=jnp.float32)
        return (out + bias) * scale_val

    pout_ref[...] = emb(pts_ref[...], P_ROWS, scale_ref[1])
    nout_ref[...] = emb(nts_ref[...], N_ROWS, scale_ref[0])


_NW = 32
_BPW = NUM_IDX // _NW
_CHUNK = 256
_NBUF = 2
_NCH = _BPW // _CHUNK
_NSUB = _CHUNK // GATHER_W
_D2 = 2 * D_REL


def _sc_gather(table_pairs, idx_flat):
    mesh = plsc.VectorSubcoreMesh(core_axis_name="c", subcore_axis_name="s")

    @pl.kernel(
        out_type=jax.ShapeDtypeStruct((NUM_IDX, _D2), jnp.float32),
        mesh=mesh,
        scratch_types=[
            pltpu.VMEM((_NBUF, _CHUNK), jnp.int32),
            pltpu.VMEM((_NBUF, _CHUNK, _D2), jnp.float32),
            pltpu.SemaphoreType.DMA((_NBUF,)),
            pltpu.SemaphoreType.DMA((_NBUF,)),
            pltpu.SemaphoreType.DMA((_NBUF,)),
        ],
    )
    def k(table_hbm, idx_hbm, out_hbm, idx_v, rows_v, sem_i, sem_g, sem_o):
        wid = jax.lax.axis_index("s") * 2 + jax.lax.axis_index("c")
        base = wid * _BPW

        for b in range(_NBUF):
            pltpu.async_copy(
                idx_hbm.at[pl.ds(base + b * _CHUNK, _CHUNK)],
                idx_v.at[b], sem_i.at[b])

        @pl.loop(0, _NCH, step=_NBUF)
        def _(ch0):
            for b in range(_NBUF):
                ch = ch0 + b
                row0 = base + ch * _CHUNK
                pltpu.make_async_copy(
                    idx_hbm.at[pl.ds(0, _CHUNK)], idx_v.at[b],
                    sem_i.at[b]).wait()

                @pl.when(ch >= _NBUF)
                def _():
                    pltpu.make_async_copy(
                        rows_v.at[b],
                        out_hbm.at[pl.ds(0, _CHUNK)],
                        sem_o.at[b]).wait()

                for s in range(_NSUB):
                    sl = pl.ds(s * GATHER_W, GATHER_W)
                    pltpu.async_copy(
                        table_hbm.at[idx_v.at[b, sl]],
                        rows_v.at[b, sl], sem_g.at[b])
                pltpu.make_async_copy(
                    table_hbm.at[pl.ds(0, _CHUNK)], rows_v.at[b],
                    sem_g.at[b]).wait()

                pltpu.async_copy(
                    rows_v.at[b],
                    out_hbm.at[pl.ds(row0, _CHUNK)],
                    sem_o.at[b])

                @pl.when(ch + _NBUF < _NCH)
                def _():
                    pltpu.async_copy(
                        idx_hbm.at[pl.ds(row0 + _NBUF * _CHUNK, _CHUNK)],
                        idx_v.at[b], sem_i.at[b])

        for b in range(_NBUF):
            pltpu.make_async_copy(
                rows_v.at[b],
                out_hbm.at[pl.ds(0, _CHUNK)],
                sem_o.at[b]).wait()

    return k(table_pairs, idx_flat)


def kernel(price_timestamps, news_timestamps, month_table, weekday_table,
           hour_table, minute_table, session_table, relpos_table, W_proj,
           b_proj, modality_scale):
    rel_idx = pl.pallas_call(
        _relidx_body,
        out_shape=jax.ShapeDtypeStruct((P_ROWS, N_ROWS), jnp.int32),
    )(price_timestamps, news_timestamps.T)

    table_pairs = jnp.concatenate([relpos_table[:-1], relpos_table[1:]], axis=1)
    gathered = _sc_gather(table_pairs, rel_idx.reshape(NUM_IDX))
    relpos = gathered[:, :D_REL].reshape(P_ROWS, N_ROWS, D_REL)

    bdiag = jnp.zeros((_COMB_PAD, 5 * D8), jnp.float32)
    bdiag = jax.lax.dynamic_update_slice(bdiag, month_table, (_OFF_M, 0))
    bdiag = jax.lax.dynamic_update_slice(bdiag, weekday_table, (_OFF_W, D8))
    bdiag = jax.lax.dynamic_update_slice(bdiag, hour_table, (_OFF_H, 2 * D8))
    bdiag = jax.lax.dynamic_update_slice(bdiag, minute_table, (_OFF_MIN, 3 * D8))
    bdiag = jax.lax.dynamic_update_slice(bdiag, session_table, (_OFF_S, 4 * D8))

    price_emb, news_emb = pl.pallas_call(
        _embed_body,
        out_shape=[
            jax.ShapeDtypeStruct((P_ROWS, D_MODEL), jnp.float32),
            jax.ShapeDtypeStruct((N_ROWS, D_MODEL), jnp.float32),
        ],
        in_specs=[
            pl.BlockSpec(memory_space=pltpu.VMEM),
            pl.BlockSpec(memory_space=pltpu.VMEM),
            pl.BlockSpec(memory_space=pltpu.VMEM),
            pl.BlockSpec(memory_space=pltpu.VMEM),
            pl.BlockSpec(memory_space=pltpu.VMEM),
            pl.BlockSpec(memory_space=pltpu.SMEM),
        ],
    )(price_timestamps, news_timestamps, bdiag, W_proj,
      b_proj.reshape(1, D_MODEL), modality_scale)

    return (price_emb, news_emb, relpos)


# speedup vs baseline: 35.2927x; 35.2927x over previous
import jax
import jax.numpy as jnp
from jax.experimental import pallas as pl
from jax.experimental.pallas import tpu as pltpu
from jax.experimental.pallas import tpu_sc as plsc

P_ROWS = 4096
N_ROWS = 200
D_MODEL = 256
D8 = D_MODEL // 8
D_REL = 64
NUM_IDX = P_ROWS * N_ROWS
GATHER_W = 128

_OFF_M, _OFF_W, _OFF_H, _OFF_MIN, _OFF_S = 0, 12, 17, 41, 101
_COMB = 105
_COMB_PAD = 128


def _relidx_body(pts_ref, nts_t_ref, out_ref):
    mb = pts_ref[:, 2:3] * 60 + pts_ref[:, 3:4]
    ma = nts_t_ref[2:3, :] * 60 + nts_t_ref[3:4, :]
    out_ref[...] = jnp.clip(mb - ma, -500, 500) + 500


def _session_col(hour, minute):
    t = hour * 60 + minute
    return jnp.where(t < 4 * 60, 0,
           jnp.where(t < 9 * 60 + 30, 1,
           jnp.where(t < 16 * 60, 2,
           jnp.where(t < 20 * 60, 3, 0))))


def _embed_body(pts_ref, nts_ref, bdiag_ref, w_ref, b_ref, scale_ref,
                pout_ref, nout_ref):
    bdiag = bdiag_ref[...]
    w = w_ref[...]
    bias = b_ref[...]

    def emb(ts, nrows, scale_val):
        cm = ts[:, 0:1] - 1 + _OFF_M
        cw = ts[:, 1:2] + _OFF_W
        ch = ts[:, 2:3] + _OFF_H
        cmin = ts[:, 3:4] + _OFF_MIN
        cs = _session_col(ts[:, 2:3], ts[:, 3:4]) + _OFF_S
        col = jax.lax.broadcasted_iota(jnp.int32, (nrows, _COMB_PAD), 1)
        h = ((col == cm).astype(jnp.float32)
             + (col == cw).astype(jnp.float32)
             + (col == ch).astype(jnp.float32)
             + (col == cmin).astype(jnp.float32)
             + (col == cs).astype(jnp.float32))
        feats = jnp.dot(h, bdiag, preferred_element_type=jnp.float32)
        out = jnp.dot(feats, w, preferred_element_type=jnp.float32)
        return (out + bias) * scale_val

    pout_ref[...] = emb(pts_ref[...], P_ROWS, scale_ref[1])
    nout_ref[...] = emb(nts_ref[...], N_ROWS, scale_ref[0])


_NW = 32
_BPW = NUM_IDX // _NW
_CHUNK = 256
_NBUF = 2
_NCH = _BPW // _CHUNK
_TROWS = 1001
_L = 16


def _sc_gather(table_flat, idx_flat):
    mesh = plsc.VectorSubcoreMesh(core_axis_name="c", subcore_axis_name="s")

    @pl.kernel(
        out_type=jax.ShapeDtypeStruct((NUM_IDX * D_REL,), jnp.float32),
        mesh=mesh,
        scratch_types=[
            pltpu.VMEM((_TROWS * D_REL,), jnp.float32),
            pltpu.VMEM((_NBUF, _CHUNK), jnp.int32),
            pltpu.VMEM((_NBUF, _CHUNK * D_REL), jnp.float32),
            pltpu.SemaphoreType.DMA((_NBUF,)),
            pltpu.SemaphoreType.DMA((_NBUF,)),
        ],
        compiler_params=pltpu.CompilerParams(
            use_tc_tiling_on_sc=False, needs_layout_passes=False),
    )
    def k(table_hbm, idx_hbm, out_hbm, table_v, idx_v, rows_v, sem_i, sem_o):
        wid = jax.lax.axis_index("s") * 2 + jax.lax.axis_index("c")
        base = wid * _BPW

        pltpu.sync_copy(table_hbm.at[pl.ds(0, _TROWS * D_REL)], table_v)

        for b in range(_NBUF):
            pltpu.async_copy(
                idx_hbm.at[pl.ds(base + b * _CHUNK, _CHUNK)],
                idx_v.at[b], sem_i.at[b])

        iota = jax.lax.iota(jnp.int32, _L)

        @pl.loop(0, _NCH, step=_NBUF)
        def _(ch0):
            for b in range(_NBUF):
                ch = ch0 + b
                row0 = base + ch * _CHUNK
                pltpu.make_async_copy(
                    idx_hbm.at[pl.ds(0, _CHUNK)], idx_v.at[b],
                    sem_i.at[b]).wait()

                @pl.when(ch >= _NBUF)
                def _():
                    pltpu.make_async_copy(
                        rows_v.at[b], out_hbm.at[pl.ds(0, _CHUNK * D_REL)],
                        sem_o.at[b]).wait()

                @pl.loop(0, _CHUNK // _L)
                def _(g):
                    rvec = idx_v[b, pl.ds(g * _L, _L)]
                    rbase = rvec * D_REL
                    obase = (g * _L + iota) * D_REL
                    colv = iota
                    for _c in range(D_REL):
                        v = plsc.load_gather(table_v, [rbase + colv])
                        plsc.store_scatter(rows_v.at[b], [obase + colv], v)
                        colv = (colv + 1) & (D_REL - 1)

                pltpu.async_copy(
                    rows_v.at[b],
                    out_hbm.at[pl.ds(row0 * D_REL, _CHUNK * D_REL)],
                    sem_o.at[b])

                @pl.when(ch + _NBUF < _NCH)
                def _():
                    pltpu.async_copy(
                        idx_hbm.at[pl.ds(row0 + _NBUF * _CHUNK, _CHUNK)],
                        idx_v.at[b], sem_i.at[b])

        for b in range(_NBUF):
            pltpu.make_async_copy(
                rows_v.at[b], out_hbm.at[pl.ds(0, _CHUNK * D_REL)],
                sem_o.at[b]).wait()

    return k(table_flat, idx_flat)


def kernel(price_timestamps, news_timestamps, month_table, weekday_table,
           hour_table, minute_table, session_table, relpos_table, W_proj,
           b_proj, modality_scale):
    rel_idx = pl.pallas_call(
        _relidx_body,
        out_shape=jax.ShapeDtypeStruct((P_ROWS, N_ROWS), jnp.int32),
    )(price_timestamps, news_timestamps.T)

    gathered = _sc_gather(relpos_table.reshape(-1), rel_idx.reshape(NUM_IDX))
    relpos = gathered.reshape(P_ROWS, N_ROWS, D_REL)

    bdiag = jnp.zeros((_COMB_PAD, 5 * D8), jnp.float32)
    bdiag = jax.lax.dynamic_update_slice(bdiag, month_table, (_OFF_M, 0))
    bdiag = jax.lax.dynamic_update_slice(bdiag, weekday_table, (_OFF_W, D8))
    bdiag = jax.lax.dynamic_update_slice(bdiag, hour_table, (_OFF_H, 2 * D8))
    bdiag = jax.lax.dynamic_update_slice(bdiag, minute_table, (_OFF_MIN, 3 * D8))
    bdiag = jax.lax.dynamic_update_slice(bdiag, session_table, (_OFF_S, 4 * D8))

    price_emb, news_emb = pl.pallas_call(
        _embed_body,
        out_shape=[
            jax.ShapeDtypeStruct((P_ROWS, D_MODEL), jnp.float32),
            jax.ShapeDtypeStruct((N_ROWS, D_MODEL), jnp.float32),
        ],
        in_specs=[
            pl.BlockSpec(memory_space=pltpu.VMEM),
            pl.BlockSpec(memory_space=pltpu.VMEM),
            pl.BlockSpec(memory_space=pltpu.VMEM),
            pl.BlockSpec(memory_space=pltpu.VMEM),
            pl.BlockSpec(memory_space=pltpu.VMEM),
            pl.BlockSpec(memory_space=pltpu.SMEM),
        ],
    )(price_timestamps, news_timestamps, bdiag, W_proj,
      b_proj.reshape(1, D_MODEL), modality_scale)

    return (price_emb, news_emb, relpos)
